# Initial kernel scaffold; baseline (speedup 1.0000x reference)
import jax
import jax.numpy as jnp
from jax.experimental import pallas as pl
from jax.experimental.pallas import tpu as pltpu

HN, DD, N1, N2, Q1V, Q2V, MV = 8, 32, 128, 128, 21, 21, 1024
LAMBD = 0.001
AP = 32
MB = 128
NB = MV // MB


def _sf_reg_kernel(q_ref, k_ref, vfl_ref, sf_ref, reg_ref):
    sfs = []
    for h in range(HN):
        e = jax.lax.dot_general(
            q_ref[h], k_ref[h], (((0,), (0,)), ((), ())),
            preferred_element_type=jnp.float32)
        mx = jnp.max(e, axis=1, keepdims=True)
        ex = jnp.exp(e - mx)
        sf = ex / jnp.sum(ex, axis=1, keepdims=True)
        sf_ref[h] = sf
        sfs.append(sf)
    s2 = jnp.stack(sfs, axis=0).reshape(HN, N1 * N2)
    m_mat = jax.lax.dot_general(
        s2, s2, (((1,), (1,)), ((), ())),
        preferred_element_type=jnp.float32)
    vfl = vfl_ref[...]
    vvt = jax.lax.dot_general(
        vfl, vfl, (((1,), (1,)), ((), ())),
        preferred_element_type=jnp.float32)
    reg_ref[0, 0] = LAMBD * jnp.sum(m_mat * vvt)


def _loss_kernel(sf_ref, z1_ref, z2_ref, w_ref, vc_ref, reg_ref, out_ref):
    b = pl.program_id(0)
    iota_a = jax.lax.broadcasted_iota(jnp.int32, (N2, MB, AP), 2)

    o2 = (z2_ref[...][:, :, None] == iota_a).astype(jnp.float32)
    dall = jax.lax.dot_general(
        o2.reshape(N2 * MB, AP), vc_ref[...], (((1,), (0,)), ((), ())),
        preferred_element_type=jnp.float32)

    lt = jnp.zeros((N1, MB * AP), jnp.float32)
    for h in range(HN):
        dh = dall[:, h * AP:(h + 1) * AP].reshape(N2, MB * AP)
        lt = lt + jax.lax.dot_general(
            sf_ref[h], dh, (((1,), (0,)), ((), ())),
            preferred_element_type=jnp.float32)
    lt3 = lt.reshape(N1, MB, AP)

    o1 = (z1_ref[...][:, :, None] == iota_a).astype(jnp.float32)
    ene = jnp.sum(lt3 * o1, axis=(0, 2))

    valid = iota_a < Q1V
    mx = jnp.maximum(
        jnp.max(jnp.where(valid, lt3, jnp.float32(-1e30)), axis=(0, 2)), 0.0)
    s = (jnp.sum(jnp.where(valid, jnp.exp(lt3 - mx[None, :, None]), 0.0),
                 axis=(0, 2))
         + jnp.float32(N1 - Q1V) * jnp.exp(-mx))
    logzf = mx + jnp.log(s)

    part = jnp.sum(w_ref[0, :] * (ene - logzf))

    @pl.when(b == 0)
    def _():
        out_ref[0, 0] = reg_ref[0, 0] - part

    @pl.when(b != 0)
    def _():
        out_ref[0, 0] = out_ref[0, 0] - part


def kernel(Z1, Z2, weights, Q, K, V):
    Z1 = Z1.astype(jnp.int32)
    Z2 = Z2.astype(jnp.int32)
    w = weights.astype(jnp.float32).reshape(1, MV)
    vpad = jnp.pad(V, ((0, 0), (0, AP - Q1V), (0, AP - Q2V)))
    vc = jnp.transpose(vpad, (2, 0, 1)).reshape(AP, HN * AP)
    vfl = jnp.pad(V.reshape(HN, Q1V * Q2V), ((0, 0), (0, 512 - Q1V * Q2V)))

    sf, reg = pl.pallas_call(
        _sf_reg_kernel,
        out_shape=(
            jax.ShapeDtypeStruct((HN, N1, N2), jnp.float32),
            jax.ShapeDtypeStruct((1, 1), jnp.float32),
        ),
    )(Q, K, vfl)

    out = pl.pallas_call(
        _loss_kernel,
        grid=(NB,),
        in_specs=[
            pl.BlockSpec((HN, N1, N2), lambda b: (0, 0, 0)),
            pl.BlockSpec((N1, MB), lambda b: (0, b)),
            pl.BlockSpec((N2, MB), lambda b: (0, b)),
            pl.BlockSpec((1, MB), lambda b: (0, b)),
            pl.BlockSpec((AP, HN * AP), lambda b: (0, 0)),
            pl.BlockSpec((1, 1), lambda b: (0, 0)),
        ],
        out_specs=pl.BlockSpec((1, 1), lambda b: (0, 0)),
        out_shape=jax.ShapeDtypeStruct((1, 1), jnp.float32),
    )(sf, Z1, Z2, w, vc, reg)
    return out[0, 0]


# speedup vs baseline: 35281.3340x; 35281.3340x over previous
import jax
import jax.numpy as jnp
from jax.experimental import pallas as pl
from jax.experimental.pallas import tpu as pltpu

HN, DD, N1, N2, Q1V, Q2V, MV = 8, 32, 128, 128, 21, 21, 1024
LAMBD = 0.001
BP = 128
MB = 128
NB = MV // MB


def _sf_reg_kernel(q_ref, k_ref, vfl_ref, sfw_ref, reg_ref):
    sfs = []
    for h in range(HN):
        e = jax.lax.dot_general(
            q_ref[h], k_ref[h], (((0,), (0,)), ((), ())),
            preferred_element_type=jnp.float32)
        mx = jnp.max(e, axis=1, keepdims=True)
        ex = jnp.exp(e - mx)
        sf = ex / jnp.sum(ex, axis=1, keepdims=True)
        sfw_ref[:, h * N2:(h + 1) * N2] = sf
        sfs.append(sf)
    reg = jnp.float32(0.0)
    for h in range(HN):
        vh = vfl_ref[h:h + 1, :]
        for k in range(h, HN):
            mult = jnp.float32(2.0 if k != h else 1.0)
            s_hk = jnp.sum(sfs[h] * sfs[k])
            v_hk = jnp.sum(vh * vfl_ref[k:k + 1, :])
            reg = reg + mult * s_hk * v_hk
    reg_ref[...] = (LAMBD * reg).reshape(1, 1)


def _loss_kernel(sfw_ref, z1_ref, z2_ref, w_ref, v2_ref, reg_ref, out_ref):
    b = pl.program_id(0)
    z1 = z1_ref[...]
    z2 = z2_ref[...]
    sfw = sfw_ref[...]

    lts = []
    for a in range(Q1V):
        dhs = []
        for h in range(HN):
            row = jnp.broadcast_to(v2_ref[h * 32 + a:h * 32 + a + 1, :],
                                   (N2, BP))
            dhs.append(jnp.take_along_axis(row, z2, axis=1,
                                           mode="promise_in_bounds"))
        dstack = jnp.concatenate(dhs, axis=0)
        lts.append(jax.lax.dot_general(
            sfw, dstack, (((1,), (0,)), ((), ())),
            preferred_element_type=jnp.float32))

    ene2 = jnp.zeros((N1, MB), jnp.float32)
    mx2 = jnp.full((N1, MB), -jnp.inf, jnp.float32)
    for a in range(Q1V):
        ene2 = ene2 + jnp.where(z1 == a, lts[a], 0.0)
        mx2 = jnp.maximum(mx2, lts[a])
    ene = jnp.sum(ene2, axis=0, keepdims=True)

    mx = jnp.maximum(jnp.max(mx2, axis=0, keepdims=True), 0.0)
    se2 = jnp.zeros((N1, MB), jnp.float32)
    for a in range(Q1V):
        se2 = se2 + jnp.exp(lts[a] - mx)
    s = (jnp.sum(se2, axis=0, keepdims=True)
         + jnp.float32(N1 - Q1V) * jnp.exp(-mx))
    logzf = mx + jnp.log(s)

    part = jnp.sum(w_ref[...] * (ene - logzf))

    @pl.when(b == 0)
    def _():
        out_ref[...] = reg_ref[...] - part.reshape(1, 1)

    @pl.when(b != 0)
    def _():
        out_ref[...] = out_ref[...] - part.reshape(1, 1)


def kernel(Z1, Z2, weights, Q, K, V):
    Z1 = Z1.astype(jnp.int32)
    Z2 = Z2.astype(jnp.int32)
    w = weights.astype(jnp.float32).reshape(1, MV)
    v2 = jnp.pad(V, ((0, 0), (0, 32 - Q1V), (0, BP - Q2V))).reshape(HN * 32, BP)
    vfl = jnp.pad(V.reshape(HN, Q1V * Q2V), ((0, 0), (0, 512 - Q1V * Q2V)))

    sfw, reg = pl.pallas_call(
        _sf_reg_kernel,
        out_shape=(
            jax.ShapeDtypeStruct((N1, HN * N2), jnp.float32),
            jax.ShapeDtypeStruct((1, 1), jnp.float32),
        ),
    )(Q, K, vfl)

    out = pl.pallas_call(
        _loss_kernel,
        grid=(NB,),
        in_specs=[
            pl.BlockSpec((N1, HN * N2), lambda b: (0, 0)),
            pl.BlockSpec((N1, MB), lambda b: (0, b)),
            pl.BlockSpec((N2, MB), lambda b: (0, b)),
            pl.BlockSpec((1, MB), lambda b: (0, b)),
            pl.BlockSpec((HN * 32, BP), lambda b: (0, 0)),
            pl.BlockSpec((1, 1), lambda b: (0, 0)),
        ],
        out_specs=pl.BlockSpec((1, 1), lambda b: (0, 0)),
        out_shape=jax.ShapeDtypeStruct((1, 1), jnp.float32),
    )(sfw, Z1, Z2, w, v2, reg)
    return out[0, 0]


# speedup vs baseline: 41073.2159x; 1.1642x over previous
import jax
import jax.numpy as jnp
from jax.experimental import pallas as pl
from jax.experimental.pallas import tpu as pltpu

HN, DD, N1, N2, Q1V, Q2V, MV = 8, 32, 128, 128, 21, 21, 1024
LAMBD = 0.001
BP = 128
MB = 256
NB = MV // MB


def _fused_kernel(q_ref, k_ref, vfl_ref, z1_ref, z2_ref, w_ref, v2_ref,
                  out_ref, sfw_ref, reg_ref):
    b = pl.program_id(0)

    @pl.when(b == 0)
    def _():
        sfs = []
        for h in range(HN):
            e = jax.lax.dot_general(
                q_ref[h], k_ref[h], (((0,), (0,)), ((), ())),
                preferred_element_type=jnp.float32)
            mx = jnp.max(e, axis=1, keepdims=True)
            ex = jnp.exp(e - mx)
            sf = ex / jnp.sum(ex, axis=1, keepdims=True)
            sfw_ref[:, h * N2:(h + 1) * N2] = sf
            sfs.append(sf)
        reg = jnp.float32(0.0)
        for h in range(HN):
            vh = vfl_ref[h:h + 1, :]
            for k in range(h, HN):
                mult = jnp.float32(2.0 if k != h else 1.0)
                s_hk = jnp.sum(sfs[h] * sfs[k])
                v_hk = jnp.sum(vh * vfl_ref[k:k + 1, :])
                reg = reg + mult * s_hk * v_hk
        reg_ref[...] = (LAMBD * reg).reshape(1, 1)

    z1 = z1_ref[...]
    z2 = z2_ref[...]
    sfw = sfw_ref[...]

    lts = []
    for a in range(Q1V):
        lt = None
        for h in range(HN):
            row = jnp.broadcast_to(v2_ref[h * 32 + a:h * 32 + a + 1, :],
                                   (N2, BP))
            dh = jnp.take_along_axis(row, z2, axis=1,
                                     mode="promise_in_bounds")
            p = jax.lax.dot_general(
                sfw[:, h * N2:(h + 1) * N2], dh, (((1,), (0,)), ((), ())),
                preferred_element_type=jnp.float32)
            lt = p if lt is None else lt + p
        lts.append(lt)

    ene2 = jnp.zeros((N1, MB), jnp.float32)
    mx2 = jnp.full((N1, MB), -jnp.inf, jnp.float32)
    for a in range(Q1V):
        ene2 = ene2 + jnp.where(z1 == a, lts[a], 0.0)
        mx2 = jnp.maximum(mx2, lts[a])
    ene = jnp.sum(ene2, axis=0, keepdims=True)

    mx = jnp.maximum(jnp.max(mx2, axis=0, keepdims=True), 0.0)
    se2 = jnp.zeros((N1, MB), jnp.float32)
    for a in range(Q1V):
        se2 = se2 + jnp.exp(lts[a] - mx)
    s = (jnp.sum(se2, axis=0, keepdims=True)
         + jnp.float32(N1 - Q1V) * jnp.exp(-mx))
    logzf = mx + jnp.log(s)

    part = jnp.sum(w_ref[...] * (ene - logzf))

    @pl.when(b == 0)
    def _():
        out_ref[...] = reg_ref[...] - part.reshape(1, 1)

    @pl.when(b != 0)
    def _():
        out_ref[...] = out_ref[...] - part.reshape(1, 1)


def kernel(Z1, Z2, weights, Q, K, V):
    Z1 = Z1.astype(jnp.int32)
    Z2 = Z2.astype(jnp.int32)
    w = weights.astype(jnp.float32).reshape(1, MV)
    v2 = jnp.pad(V, ((0, 0), (0, 32 - Q1V), (0, BP - Q2V))).reshape(
        HN * 32, BP)
    vfl = jnp.pad(V.reshape(HN, Q1V * Q2V), ((0, 0), (0, 512 - Q1V * Q2V)))

    out = pl.pallas_call(
        _fused_kernel,
        grid=(NB,),
        in_specs=[
            pl.BlockSpec((HN, DD, N1), lambda b: (0, 0, 0)),
            pl.BlockSpec((HN, DD, N2), lambda b: (0, 0, 0)),
            pl.BlockSpec((HN, 512), lambda b: (0, 0)),
            pl.BlockSpec((N1, MB), lambda b: (0, b)),
            pl.BlockSpec((N2, MB), lambda b: (0, b)),
            pl.BlockSpec((1, MB), lambda b: (0, b)),
            pl.BlockSpec((HN * 32, BP), lambda b: (0, 0)),
        ],
        out_specs=pl.BlockSpec((1, 1), lambda b: (0, 0)),
        out_shape=jax.ShapeDtypeStruct((1, 1), jnp.float32),
        scratch_shapes=[
            pltpu.VMEM((N1, HN * N2), jnp.float32),
            pltpu.VMEM((1, 1), jnp.float32),
        ],
    )(Q, K, vfl, Z1, Z2, w, v2)
    return out[0, 0]
